# baseline (speedup 1.0000x reference)
import functools

import jax
import jax.numpy as jnp
from jax import lax
from jax.experimental import pallas as pl
from jax.experimental.pallas import tpu as pltpu
from jax.experimental.pallas import tpu_sc as plsc

NC = 2
NS = 16
NW = NC * NS
LANES = 16
CHUNK = 125
QCH = 16
ACHUNK = 100
AQ = 20
NBUF = 3
RPT = 640
N_PAD = NS * RPT


def _sc_accumulate(h, src_r, dst_r):
  N, D = h.shape
  _, nq, _, _ = src_r.shape

  scratch = dict(
      acc_sh=pltpu.VMEM_SHARED((N_PAD, D), jnp.float32),
      sidx_v=pltpu.VMEM((AQ, ACHUNK), jnp.int32),
      didx_v=pltpu.VMEM((AQ, ACHUNK), jnp.int32),
      rows_v=[pltpu.VMEM((ACHUNK, D), jnp.float32) for _ in range(NBUF)],
      gsem=[pltpu.SemaphoreType.DMA for _ in range(NBUF)],
      ssem=[pltpu.SemaphoreType.DMA for _ in range(NBUF)],
  )

  mesh = plsc.VectorSubcoreMesh(core_axis_name="c", subcore_axis_name="s")

  def body(h_hbm, src_hbm, dst_hbm, parts_hbm, acc_sh, sidx_v, didx_v,
           rows, gsems, ssems):
    c = lax.axis_index("c")
    s = lax.axis_index("s")
    w = c * NS + s
    rows0_v = rows[0]

    zf = jnp.zeros((LANES,), jnp.float32)
    vecs_per_row = D // LANES

    def zloop(i, _):
      rows0_v[i // vecs_per_row, pl.ds((i % vecs_per_row) * LANES, LANES)] = zf
      return 0
    lax.fori_loop(0, ACHUNK * vecs_per_row, zloop, 0)

    base = s * RPT
    nfull = RPT // ACHUNK
    for t in range(nfull):
      pltpu.sync_copy(rows0_v, acc_sh.at[pl.ds(base + t * ACHUNK, ACHUNK)])
    rem = RPT - nfull * ACHUNK
    if rem:
      pltpu.sync_copy(rows0_v.at[pl.ds(0, rem)],
                      acc_sh.at[pl.ds(base + nfull * ACHUNK, rem)])
    plsc.subcore_barrier()

    def slab_body(q, _):
      pltpu.sync_copy(src_hbm.at[w, q], sidx_v)
      pltpu.sync_copy(dst_hbm.at[w, q], didx_v)
      gd = [None] * NBUF
      sd = [None] * NBUF
      for b in range(NBUF):
        gd[b] = pltpu.async_copy(h_hbm.at[sidx_v.at[b]], rows[b], gsems[b])
      for qi in range(AQ):
        p = qi % NBUF
        if qi >= 1 and qi + NBUF - 1 < AQ:
          q2 = qi + NBUF - 1
          pp = q2 % NBUF
          sd[pp].wait()
          gd[pp] = pltpu.async_copy(h_hbm.at[sidx_v.at[q2]], rows[pp],
                                    gsems[pp])
        gd[p].wait()
        sd[p] = pltpu.async_copy(rows[p], acc_sh.at[didx_v.at[qi]], ssems[p],
                                 add=True)
      for b in range(NBUF):
        if sd[(AQ - NBUF + b) % NBUF] is not None and AQ - NBUF + b >= 0:
          pass
      sd[(AQ - 3) % NBUF].wait()
      sd[(AQ - 2) % NBUF].wait()
      sd[(AQ - 1) % NBUF].wait()
      return 0
    lax.fori_loop(0, nq, slab_body, 0)

    plsc.subcore_barrier()
    pltpu.sync_copy(acc_sh.at[pl.ds(base, RPT)], parts_hbm.at[c, s])

  kern = pl.kernel(
      body,
      out_type=jax.ShapeDtypeStruct((NC, NS, RPT, D), jnp.float32),
      mesh=mesh,
      scratch_types=list(scratch.values()),
  )
  return kern(h, src_r, dst_r)


def _sc_degrees(dst_g, N):
  _, G, _ = dst_g.shape

  scratch = dict(
      didx_v=pltpu.VMEM((G, LANES), jnp.int32),
      deg_v=pltpu.VMEM((N,), jnp.float32),
      one_v=pltpu.VMEM((LANES,), jnp.float32),
  )

  mesh = plsc.VectorSubcoreMesh(core_axis_name="c", subcore_axis_name="s")

  def body(dst_hbm, deg_hbm, didx_v, deg_v, one_v):
    c = lax.axis_index("c")
    s = lax.axis_index("s")
    w = c * NS + s

    zf = jnp.zeros((LANES,), jnp.float32)

    def zloop(i, _):
      deg_v[pl.ds(i * LANES, LANES)] = zf
      return 0
    lax.fori_loop(0, N // LANES, zloop, 0)

    pltpu.sync_copy(dst_hbm.at[w], didx_v)
    one_v[pl.ds(0, LANES)] = jnp.ones((LANES,), jnp.float32)

    def gloop(g, _):
      idx = didx_v[g, pl.ds(0, LANES)]
      plsc.addupdate_scatter(deg_v, [idx], one_v[pl.ds(0, LANES)])
      return 0
    lax.fori_loop(0, G, gloop, 0)

    pltpu.sync_copy(deg_v, deg_hbm.at[w])

  kern = pl.kernel(
      body,
      out_type=jax.ShapeDtypeStruct((NW, N), jnp.float32),
      mesh=mesh,
      scratch_types=list(scratch.values()),
      compiler_params=pltpu.CompilerParams(needs_layout_passes=False),
  )
  return kern(dst_g)


def _tc_layer_body(p0, p1, h, dp, w_ref, b_ref, out):
  dsum = jnp.sum(dp[...], axis=1, keepdims=True)
  inv = 1.0 / (dsum + 1.0)
  hn = (p0[...] + p1[...] + h[...]) * inv
  acc = jnp.dot(hn, w_ref[...], preferred_element_type=jnp.float32)
  out[...] = jnp.maximum(acc + b_ref[...], 0.0)


def _tc_layer(parts, h, deg_p, W, b):
  N, D = h.shape
  BN = 2000
  grid = (N // BN,)
  row_spec = pl.BlockSpec((BN, D), lambda i: (i, 0))
  return pl.pallas_call(
      _tc_layer_body,
      grid=grid,
      in_specs=[row_spec, row_spec, row_spec,
                pl.BlockSpec((BN, NW), lambda i: (i, 0)),
                pl.BlockSpec((D, D), lambda i: (0, 0)),
                pl.BlockSpec((1, D), lambda i: (0, 0))],
      out_specs=row_spec,
      out_shape=jax.ShapeDtypeStruct((N, D), jnp.float32),
  )(parts[0], parts[1], h, deg_p, W, b.reshape(1, D))


@jax.jit
def kernel(x, edge_index, W1, b1, W2, b2):
  N, D = x.shape
  E = edge_index.shape[1]
  nq = E // (NW * AQ * ACHUNK)
  src_r = edge_index[0].reshape(NW, nq, AQ, ACHUNK)
  dst_r = edge_index[1].reshape(NW, nq, AQ, ACHUNK)
  dst_g = edge_index[1].reshape(NW, E // (NW * LANES), LANES)

  deg_p = _sc_degrees(dst_g, N).T
  parts1 = _sc_accumulate(x, src_r, dst_r).reshape(NC, N_PAD, D)
  h1 = _tc_layer(parts1, x, deg_p, W1, b1)
  parts2 = _sc_accumulate(h1, src_r, dst_r).reshape(NC, N_PAD, D)
  h2 = _tc_layer(parts2, h1, deg_p, W2, b2)
  return h2


# speedup vs baseline: 1.0136x; 1.0136x over previous
import functools

import jax
import jax.numpy as jnp
from jax import lax
from jax.experimental import pallas as pl
from jax.experimental.pallas import tpu as pltpu
from jax.experimental.pallas import tpu_sc as plsc

NC = 2
NS = 16
NW = NC * NS
LANES = 16
CHUNK = 125
QCH = 16
ACHUNK = 80
AQ = 25
NBUF = 4
RPT = 640
N_PAD = NS * RPT


def _sc_accumulate(h, src_r, dst_r):
  N, D = h.shape
  _, nq, _, _ = src_r.shape

  scratch = dict(
      acc_sh=pltpu.VMEM_SHARED((N_PAD, D), jnp.float32),
      sidx_v=pltpu.VMEM((AQ, ACHUNK), jnp.int32),
      didx_v=pltpu.VMEM((AQ, ACHUNK), jnp.int32),
      rows_v=[pltpu.VMEM((ACHUNK, D), jnp.float32) for _ in range(NBUF)],
      gsem=[pltpu.SemaphoreType.DMA for _ in range(NBUF)],
      ssem=[pltpu.SemaphoreType.DMA for _ in range(NBUF)],
  )

  mesh = plsc.VectorSubcoreMesh(core_axis_name="c", subcore_axis_name="s")

  def body(h_hbm, src_hbm, dst_hbm, parts_hbm, acc_sh, sidx_v, didx_v,
           rows, gsems, ssems):
    c = lax.axis_index("c")
    s = lax.axis_index("s")
    w = c * NS + s
    rows0_v = rows[0]

    zf = jnp.zeros((LANES,), jnp.float32)
    vecs_per_row = D // LANES

    def zloop(i, _):
      rows0_v[i // vecs_per_row, pl.ds((i % vecs_per_row) * LANES, LANES)] = zf
      return 0
    lax.fori_loop(0, ACHUNK * vecs_per_row, zloop, 0)

    base = s * RPT
    nfull = RPT // ACHUNK
    for t in range(nfull):
      pltpu.sync_copy(rows0_v, acc_sh.at[pl.ds(base + t * ACHUNK, ACHUNK)])
    rem = RPT - nfull * ACHUNK
    if rem:
      pltpu.sync_copy(rows0_v.at[pl.ds(0, rem)],
                      acc_sh.at[pl.ds(base + nfull * ACHUNK, rem)])
    plsc.subcore_barrier()

    def slab_body(q, _):
      pltpu.sync_copy(src_hbm.at[w, q], sidx_v)
      pltpu.sync_copy(dst_hbm.at[w, q], didx_v)
      gd = [None] * NBUF
      sd = [None] * NBUF
      for b in range(NBUF):
        gd[b] = pltpu.async_copy(h_hbm.at[sidx_v.at[b]], rows[b], gsems[b])
      for qi in range(AQ):
        p = qi % NBUF
        if qi >= 1 and qi + NBUF - 1 < AQ:
          q2 = qi + NBUF - 1
          pp = q2 % NBUF
          sd[pp].wait()
          gd[pp] = pltpu.async_copy(h_hbm.at[sidx_v.at[q2]], rows[pp],
                                    gsems[pp])
        gd[p].wait()
        sd[p] = pltpu.async_copy(rows[p], acc_sh.at[didx_v.at[qi]], ssems[p],
                                 add=True)
      for b in range(1, NBUF + 1):
        sd[(AQ - b) % NBUF].wait()
      return 0
    lax.fori_loop(0, nq, slab_body, 0)

    plsc.subcore_barrier()
    pltpu.sync_copy(acc_sh.at[pl.ds(base, RPT)], parts_hbm.at[c, s])

  kern = pl.kernel(
      body,
      out_type=jax.ShapeDtypeStruct((NC, NS, RPT, D), jnp.float32),
      mesh=mesh,
      scratch_types=list(scratch.values()),
  )
  return kern(h, src_r, dst_r)


def _sc_degrees(dst_g, N):
  _, G, _ = dst_g.shape

  scratch = dict(
      didx_v=pltpu.VMEM((G, LANES), jnp.int32),
      deg_v=pltpu.VMEM((N,), jnp.float32),
      one_v=pltpu.VMEM((LANES,), jnp.float32),
  )

  mesh = plsc.VectorSubcoreMesh(core_axis_name="c", subcore_axis_name="s")

  def body(dst_hbm, deg_hbm, didx_v, deg_v, one_v):
    c = lax.axis_index("c")
    s = lax.axis_index("s")
    w = c * NS + s

    zf = jnp.zeros((LANES,), jnp.float32)

    def zloop(i, _):
      deg_v[pl.ds(i * LANES, LANES)] = zf
      return 0
    lax.fori_loop(0, N // LANES, zloop, 0)

    pltpu.sync_copy(dst_hbm.at[w], didx_v)
    one_v[pl.ds(0, LANES)] = jnp.ones((LANES,), jnp.float32)

    def gloop(g, _):
      idx = didx_v[g, pl.ds(0, LANES)]
      plsc.addupdate_scatter(deg_v, [idx], one_v[pl.ds(0, LANES)])
      return 0
    lax.fori_loop(0, G, gloop, 0)

    pltpu.sync_copy(deg_v, deg_hbm.at[w])

  kern = pl.kernel(
      body,
      out_type=jax.ShapeDtypeStruct((NW, N), jnp.float32),
      mesh=mesh,
      scratch_types=list(scratch.values()),
      compiler_params=pltpu.CompilerParams(needs_layout_passes=False),
  )
  return kern(dst_g)


def _tc_layer_body(p0, p1, h, dp, w_ref, b_ref, out):
  dsum = jnp.sum(dp[...], axis=1, keepdims=True)
  inv = 1.0 / (dsum + 1.0)
  hn = (p0[...] + p1[...] + h[...]) * inv
  acc = jnp.dot(hn, w_ref[...], preferred_element_type=jnp.float32)
  out[...] = jnp.maximum(acc + b_ref[...], 0.0)


def _tc_layer(parts, h, deg_p, W, b):
  N, D = h.shape
  BN = 2000
  grid = (N // BN,)
  row_spec = pl.BlockSpec((BN, D), lambda i: (i, 0))
  return pl.pallas_call(
      _tc_layer_body,
      grid=grid,
      in_specs=[row_spec, row_spec, row_spec,
                pl.BlockSpec((BN, NW), lambda i: (i, 0)),
                pl.BlockSpec((D, D), lambda i: (0, 0)),
                pl.BlockSpec((1, D), lambda i: (0, 0))],
      out_specs=row_spec,
      out_shape=jax.ShapeDtypeStruct((N, D), jnp.float32),
  )(parts[0], parts[1], h, deg_p, W, b.reshape(1, D))


@jax.jit
def kernel(x, edge_index, W1, b1, W2, b2):
  N, D = x.shape
  E = edge_index.shape[1]
  nq = E // (NW * AQ * ACHUNK)
  src_r = edge_index[0].reshape(NW, nq, AQ, ACHUNK)
  dst_r = edge_index[1].reshape(NW, nq, AQ, ACHUNK)
  dst_g = edge_index[1].reshape(NW, E // (NW * LANES), LANES)

  deg_p = _sc_degrees(dst_g, N).T
  parts1 = _sc_accumulate(x, src_r, dst_r).reshape(NC, N_PAD, D)
  h1 = _tc_layer(parts1, x, deg_p, W1, b1)
  parts2 = _sc_accumulate(h1, src_r, dst_r).reshape(NC, N_PAD, D)
  h2 = _tc_layer(parts2, h1, deg_p, W2, b2)
  return h2
